# baseline (speedup 1.0000x reference)
import functools

import jax
import jax.numpy as jnp
from jax import lax
from jax.experimental import pallas as pl
from jax.experimental.pallas import tpu as pltpu
from jax.experimental.pallas import tpu_sc as plsc

_D = 64
_SCALE = 8.0
_NC, _NS = 2, 16
_NW = _NC * _NS
_CHUNK = 128
_K = 4
_SUP = _CHUNK * _K


@functools.lru_cache(maxsize=None)
def _make_emb(tot):
    per_w = tot // _NW
    nsup = per_w // _SUP
    assert per_w * _NW == tot and nsup * _SUP == per_w and nsup % 2 == 0
    chunks_per_w = per_w // _CHUNK
    mesh = plsc.VectorSubcoreMesh(core_axis_name="c", subcore_axis_name="s")

    @functools.partial(
        pl.kernel,
        out_type=jax.ShapeDtypeStruct((tot, _D), jnp.float32),
        mesh=mesh,
        compiler_params=pltpu.CompilerParams(use_tc_tiling_on_sc=False),
        scratch_types=[
            pltpu.VMEM((2, _K, _CHUNK), jnp.int32),
            pltpu.VMEM((2, _SUP, _D), jnp.float32),
            pltpu.SemaphoreType.DMA,
            pltpu.SemaphoreType.DMA,
        ],
    )
    def emb(idx_hbm, table_hbm, out_hbm, idx_v, rows_v, sem0, sem1):
        wid = lax.axis_index("s") * _NC + lax.axis_index("c")
        cbase = wid * chunks_per_w

        def fire(s, b, sem):
            c0 = cbase + s * _K
            pltpu.sync_copy(idx_hbm.at[pl.ds(c0, _K)], idx_v.at[b])
            for j in range(_K):
                pltpu.async_copy(
                    table_hbm.at[idx_v.at[b, j]],
                    rows_v.at[b, pl.ds(j * _CHUNK, _CHUNK)],
                    sem,
                )

        def drain(sem, b):
            pltpu.make_async_copy(
                table_hbm.at[pl.ds(0, _SUP)], rows_v.at[b], sem
            ).wait()

        def scale_store(s, b):
            buf = rows_v.at[b]

            @plsc.parallel_loop(0, _SUP, step=1, unroll=8)
            def _scale(r):
                for c in range(_D // 16):
                    buf[r, pl.ds(c * 16, 16)] = buf[r, pl.ds(c * 16, 16)] * _SCALE

            pltpu.sync_copy(
                buf, out_hbm.at[pl.ds((cbase + s * _K) * _CHUNK, _SUP)]
            )

        fire(0, 0, sem0)

        def body(p, carry):
            s0 = 2 * p
            fire(s0 + 1, 1, sem1)
            drain(sem0, 0)
            scale_store(s0, 0)
            fire(jnp.minimum(s0 + 2, nsup - 1), 0, sem0)
            drain(sem1, 1)
            scale_store(s0 + 1, 1)
            return carry

        lax.fori_loop(0, nsup // 2, body, 0)
        drain(sem0, 0)

    return emb


def kernel(x, table):
    b, h = x.shape
    tot = b * h
    idx2 = x.reshape(tot // _CHUNK, _CHUNK).astype(jnp.int32)
    out = _make_emb(tot)(idx2, table)
    return out.reshape(b, h, _D)
